# Initial kernel scaffold; baseline (speedup 1.0000x reference)
import jax
import jax.numpy as jnp
from jax.experimental import pallas as pl


def kernel(x, var_table, gamma, beta):
    raise NotImplementedError("write your pallas kernel here")


# speedup vs baseline: 2.1124x; 2.1124x over previous
import jax
import jax.numpy as jnp
from jax.experimental import pallas as pl

_EPS = 1e-12


def _tc_body(x_ref, vt_ref, g_ref, b_ref, o_ref):
    S = x_ref.shape[1]
    v = vt_ref[:S, :]
    mv = jnp.mean(v, axis=1, keepdims=True)
    cv = v - mv
    rowvar = jnp.mean(cv * cv, axis=1, keepdims=False)
    ng = cv * g_ref[...][None, :]
    x = x_ref[...]
    scale = x * jax.lax.rsqrt(x * x * rowvar[None, :] + _EPS)
    o_ref[...] = scale[:, :, None] * ng[None, :, :] + b_ref[...][None, None, :]


def kernel(x, var_table, gamma, beta):
    B, S = x.shape
    H = var_table.shape[1]
    BB = 64
    grid = (B // BB,)
    return pl.pallas_call(
        _tc_body,
        grid=grid,
        in_specs=[
            pl.BlockSpec((BB, S), lambda i: (i, 0)),
            pl.BlockSpec((var_table.shape[0], H), lambda i: (0, 0)),
            pl.BlockSpec((H,), lambda i: (0,)),
            pl.BlockSpec((H,), lambda i: (0,)),
        ],
        out_specs=pl.BlockSpec((BB, S, H), lambda i: (i, 0, 0)),
        out_shape=jax.ShapeDtypeStruct((B, S, H), jnp.float32),
    )(x, var_table, gamma, beta)


# speedup vs baseline: 2.2541x; 1.0671x over previous
import jax
import jax.numpy as jnp
from jax.experimental import pallas as pl

_EPS = 1e-12


def _tc_body(x_ref, vt_ref, g_ref, b_ref, o_ref):
    S = x_ref.shape[1]
    v = vt_ref[:S, :]
    mv = jnp.mean(v, axis=1, keepdims=True)
    cv = v - mv
    rowvar = jnp.mean(cv * cv, axis=1, keepdims=False)
    ng = cv * g_ref[...][None, :]
    x = x_ref[...]
    scale = x * jax.lax.rsqrt(x * x * rowvar[None, :] + _EPS)
    o_ref[...] = scale[:, :, None] * ng[None, :, :] + b_ref[...][None, None, :]


def kernel(x, var_table, gamma, beta):
    B, S = x.shape
    H = var_table.shape[1]
    BB = 128
    grid = (B // BB,)
    return pl.pallas_call(
        _tc_body,
        grid=grid,
        in_specs=[
            pl.BlockSpec((BB, S), lambda i: (i, 0)),
            pl.BlockSpec((var_table.shape[0], H), lambda i: (0, 0)),
            pl.BlockSpec((H,), lambda i: (0,)),
            pl.BlockSpec((H,), lambda i: (0,)),
        ],
        out_specs=pl.BlockSpec((BB, S, H), lambda i: (i, 0, 0)),
        out_shape=jax.ShapeDtypeStruct((B, S, H), jnp.float32),
    )(x, var_table, gamma, beta)


# speedup vs baseline: 2.3654x; 1.0494x over previous
import jax
import jax.numpy as jnp
from jax.experimental import pallas as pl
from jax.experimental.pallas import tpu as pltpu

_EPS = 1e-12
_BB = 128
_NBUF = 4


def _body(x_ref, vt_ref, g_ref, b_ref, o_hbm, buf, sem):
    B, S = x_ref.shape
    H = vt_ref.shape[1]
    n_chunks = B // _BB

    v = vt_ref[:S, :]
    mv = jnp.mean(v, axis=1, keepdims=True)
    cv = v - mv
    rowvar = jnp.mean(cv * cv, axis=1, keepdims=False)
    ng = cv * g_ref[...][None, :]
    beta = b_ref[...][None, None, :]

    def copy(i, slot):
        return pltpu.make_async_copy(
            buf.at[slot], o_hbm.at[pl.ds(i * _BB, _BB)], sem.at[slot])

    for i in range(n_chunks):
        slot = i % _NBUF
        if i >= _NBUF:
            copy(i - _NBUF, slot).wait()
        x = x_ref[pl.ds(i * _BB, _BB), :]
        scale = x * jax.lax.rsqrt(x * x * rowvar[None, :] + _EPS)
        buf[slot] = scale[:, :, None] * ng[None, :, :] + beta
        copy(i, slot).start()
    for i in range(max(0, n_chunks - _NBUF), n_chunks):
        copy(i, i % _NBUF).wait()


def kernel(x, var_table, gamma, beta):
    B, S = x.shape
    H = var_table.shape[1]
    return pl.pallas_call(
        _body,
        in_specs=[
            pl.BlockSpec(memory_space=pltpu.VMEM),
            pl.BlockSpec(memory_space=pltpu.VMEM),
            pl.BlockSpec(memory_space=pltpu.VMEM),
            pl.BlockSpec(memory_space=pltpu.VMEM),
        ],
        out_specs=pl.BlockSpec(memory_space=pl.ANY),
        out_shape=jax.ShapeDtypeStruct((B, S, H), jnp.float32),
        scratch_shapes=[
            pltpu.VMEM((_NBUF, _BB, S, H), jnp.float32),
            pltpu.SemaphoreType.DMA((_NBUF,)),
        ],
    )(x, var_table, gamma, beta)


# speedup vs baseline: 2.3701x; 1.0020x over previous
import jax
import jax.numpy as jnp
from jax.experimental import pallas as pl
from jax.experimental.pallas import tpu as pltpu

_EPS = 1e-12
_BB = 64
_NBUF = 8


def _body(x_ref, vt_ref, g_ref, b_ref, o_hbm, buf, sem):
    B, S = x_ref.shape
    H = vt_ref.shape[1]
    n_chunks = B // _BB

    v = vt_ref[:S, :]
    mv = jnp.mean(v, axis=1, keepdims=True)
    cv = v - mv
    rowvar = jnp.mean(cv * cv, axis=1, keepdims=False)
    ng = cv * g_ref[...][None, :]
    beta = b_ref[...][None, None, :]

    def copy(i, slot):
        return pltpu.make_async_copy(
            buf.at[slot], o_hbm.at[pl.ds(i * _BB, _BB)], sem.at[slot])

    for i in range(n_chunks):
        slot = i % _NBUF
        if i >= _NBUF:
            copy(i - _NBUF, slot).wait()
        x = x_ref[pl.ds(i * _BB, _BB), :]
        scale = x * jax.lax.rsqrt(x * x * rowvar[None, :] + _EPS)
        buf[slot] = scale[:, :, None] * ng[None, :, :] + beta
        copy(i, slot).start()
    for i in range(max(0, n_chunks - _NBUF), n_chunks):
        copy(i, i % _NBUF).wait()


def kernel(x, var_table, gamma, beta):
    B, S = x.shape
    H = var_table.shape[1]
    return pl.pallas_call(
        _body,
        in_specs=[
            pl.BlockSpec(memory_space=pltpu.VMEM),
            pl.BlockSpec(memory_space=pltpu.VMEM),
            pl.BlockSpec(memory_space=pltpu.VMEM),
            pl.BlockSpec(memory_space=pltpu.VMEM),
        ],
        out_specs=pl.BlockSpec(memory_space=pl.ANY),
        out_shape=jax.ShapeDtypeStruct((B, S, H), jnp.float32),
        scratch_shapes=[
            pltpu.VMEM((_NBUF, _BB, S, H), jnp.float32),
            pltpu.SemaphoreType.DMA((_NBUF,)),
        ],
    )(x, var_table, gamma, beta)
